# Initial kernel scaffold; baseline (speedup 1.0000x reference)
import jax
import jax.numpy as jnp
from jax.experimental import pallas as pl


def kernel(x, edge_index, batch, fc0_w, fc0_b, conv1_weight, conv1_u, conv1_c, conv1_bias, conv2_weight, conv2_u, conv2_c, conv2_bias, fc1_w, fc1_b):
    raise NotImplementedError("write your pallas kernel here")


# speedup vs baseline: 2.0426x; 2.0426x over previous
import functools

import jax
import jax.numpy as jnp
from jax import lax
from jax.experimental import pallas as pl
from jax.experimental.pallas import tpu as pltpu
from jax.experimental.pallas import tpu_sc as plsc

N = 50000
NP = 50176
E = 800000
HEADS = 8
NUM_GRAPHS = 16

ROW_BLK = 1024
NEG = -1e30

NUM_CORES = 2
NUM_SUBCORES = 16
CH = 80
EDGES_PER_TILE = E // NUM_SUBCORES
ROWS_PER_TILE = NP // NUM_SUBCORES
ZROWS = 112


def _tc_grid_spec(n_in, n_out):
  pass


def _stage1_body(x_ref, w0_ref, b0_ref, wa_ref, wb_ref, up_ref,
                 xwa_ref, xwb_ref, pt_ref):
  h = jnp.maximum(
      jnp.dot(x_ref[...], w0_ref[...], preferred_element_type=jnp.float32)
      + b0_ref[...], 0.0)
  xwa_ref[...] = jnp.dot(h, wa_ref[...], preferred_element_type=jnp.float32)
  xwb_ref[...] = jnp.dot(h, wb_ref[...], preferred_element_type=jnp.float32)
  pt_ref[...] = jnp.dot(h, up_ref[...], preferred_element_type=jnp.float32)


def _stage1(x, w0, b0, wa, wb, up):
  nblk = NP // ROW_BLK
  rb = lambda i: (i, 0)
  whole = lambda i: (0, 0)
  return pl.pallas_call(
      _stage1_body,
      grid=(nblk,),
      in_specs=[
          pl.BlockSpec((ROW_BLK, 128), rb),
          pl.BlockSpec((128, 16), whole),
          pl.BlockSpec((1, 16), whole),
          pl.BlockSpec((16, 128), whole),
          pl.BlockSpec((16, 128), whole),
          pl.BlockSpec((16, 16), whole),
      ],
      out_specs=[
          pl.BlockSpec((ROW_BLK, 128), rb),
          pl.BlockSpec((ROW_BLK, 128), rb),
          pl.BlockSpec((ROW_BLK, 16), rb),
      ],
      out_shape=[
          jax.ShapeDtypeStruct((NP, 128), jnp.float32),
          jax.ShapeDtypeStruct((NP, 128), jnp.float32),
          jax.ShapeDtypeStruct((NP, 16), jnp.float32),
      ],
  )(x, w0, b0, wa, wb, up)


def _stage2_body(acca_ref, accb_ref, xwa_ref, xwb_ref, deg_ref, qc_ref,
                 b_ref, wa_ref, wb_ref, up_ref, xwa2_ref, xwb2_ref, pt2_ref,
                 *, chalf, heads):
  degf = deg_ref[...] + 1.0
  inv = 1.0 / degf
  selfa = (xwa_ref[...] * qc_ref[...]).reshape(-1, heads, chalf).sum(axis=1)
  selfb = (xwb_ref[...] * qc_ref[...]).reshape(-1, heads, chalf).sum(axis=1)
  ha = jnp.maximum((acca_ref[...] + selfa) * inv + b_ref[:, :chalf], 0.0)
  hb = jnp.maximum((accb_ref[...] + selfb) * inv + b_ref[:, chalf:], 0.0)
  h = jnp.concatenate([ha, hb], axis=1)
  xwa2_ref[...] = jnp.dot(h, wa_ref[...], preferred_element_type=jnp.float32)
  xwb2_ref[...] = jnp.dot(h, wb_ref[...], preferred_element_type=jnp.float32)
  pt2_ref[...] = jnp.dot(h, up_ref[...], preferred_element_type=jnp.float32)


def _stage2(acca, accb, xwa, xwb, deg2d, qcrow, brow, w2a, w2b, u2p):
  nblk = NP // ROW_BLK
  rb = lambda i: (i, 0)
  whole = lambda i: (0, 0)
  return pl.pallas_call(
      functools.partial(_stage2_body, chalf=16, heads=HEADS),
      grid=(nblk,),
      in_specs=[
          pl.BlockSpec((ROW_BLK, 16), rb),
          pl.BlockSpec((ROW_BLK, 16), rb),
          pl.BlockSpec((ROW_BLK, 128), rb),
          pl.BlockSpec((ROW_BLK, 128), rb),
          pl.BlockSpec((ROW_BLK, 1), rb),
          pl.BlockSpec((1, 128), whole),
          pl.BlockSpec((1, 32), whole),
          pl.BlockSpec((32, 256), whole),
          pl.BlockSpec((32, 256), whole),
          pl.BlockSpec((32, 16), whole),
      ],
      out_specs=[
          pl.BlockSpec((ROW_BLK, 256), rb),
          pl.BlockSpec((ROW_BLK, 256), rb),
          pl.BlockSpec((ROW_BLK, 16), rb),
      ],
      out_shape=[
          jax.ShapeDtypeStruct((NP, 256), jnp.float32),
          jax.ShapeDtypeStruct((NP, 256), jnp.float32),
          jax.ShapeDtypeStruct((NP, 16), jnp.float32),
      ],
  )(acca, accb, xwa, xwb, deg2d, qcrow, brow, w2a, w2b, u2p)


def _stage3_body(acca_ref, accb_ref, xwa_ref, xwb_ref, deg_ref, qc_ref,
                 b_ref, batch_ref, w1_ref, b1_ref, out_ref, pool_ref, cnt_ref,
                 *, chalf, heads, nblk):
  i = pl.program_id(0)

  @pl.when(i == 0)
  def _init():
    pool_ref[...] = jnp.zeros_like(pool_ref)
    cnt_ref[...] = jnp.zeros_like(cnt_ref)

  degf = deg_ref[...] + 1.0
  inv = 1.0 / degf
  selfa = (xwa_ref[...] * qc_ref[...]).reshape(-1, heads, chalf).sum(axis=1)
  selfb = (xwb_ref[...] * qc_ref[...]).reshape(-1, heads, chalf).sum(axis=1)
  ha = jnp.maximum((acca_ref[...] + selfa) * inv + b_ref[:, :chalf], 0.0)
  hb = jnp.maximum((accb_ref[...] + selfb) * inv + b_ref[:, chalf:], 0.0)
  h = jnp.concatenate([ha, hb], axis=1)
  bb = batch_ref[...]
  for g in range(NUM_GRAPHS):
    m = (bb == g).astype(jnp.float32)
    pool_ref[g:g + 1, :] += jnp.sum(h * m, axis=0, keepdims=True)
    cnt_ref[g:g + 1, :] += jnp.sum(m)

  @pl.when(i == nblk - 1)
  def _final():
    cnt = jnp.maximum(cnt_ref[:, :1], 1.0)
    pooled = pool_ref[...] / cnt
    out_ref[...] = (
        jnp.dot(pooled, w1_ref[...], preferred_element_type=jnp.float32)
        + b1_ref[...])


def _stage3(acca, accb, xwa, xwb, deg2d, qcrow, brow, batch2d, fc1_w, fc1b):
  nblk = NP // ROW_BLK
  rb = lambda i: (i, 0)
  whole = lambda i: (0, 0)
  return pl.pallas_call(
      functools.partial(_stage3_body, chalf=32, heads=HEADS, nblk=nblk),
      grid=(nblk,),
      in_specs=[
          pl.BlockSpec((ROW_BLK, 32), rb),
          pl.BlockSpec((ROW_BLK, 32), rb),
          pl.BlockSpec((ROW_BLK, 256), rb),
          pl.BlockSpec((ROW_BLK, 256), rb),
          pl.BlockSpec((ROW_BLK, 1), rb),
          pl.BlockSpec((1, 256), whole),
          pl.BlockSpec((1, 64), whole),
          pl.BlockSpec((ROW_BLK, 1), rb),
          pl.BlockSpec((64, 16), whole),
          pl.BlockSpec((1, 16), whole),
      ],
      out_specs=pl.BlockSpec((NUM_GRAPHS, 16), whole),
      out_shape=jax.ShapeDtypeStruct((NUM_GRAPHS, 16), jnp.float32),
      scratch_shapes=[
          pltpu.VMEM((NUM_GRAPHS, 64), jnp.float32),
          pltpu.VMEM((NUM_GRAPHS, 128), jnp.float32),
      ],
  )(acca, accb, xwa, xwb, deg2d, qcrow, brow, batch2d, fc1_w, fc1b)


SCOLS = 32


def _make_edge_kernel(chalf, k_cols, with_deg):
  nv = chalf // 16
  nchunk = EDGES_PER_TILE // CH

  mesh = plsc.VectorSubcoreMesh(core_axis_name="c", subcore_axis_name="s")

  out_type = [
      jax.ShapeDtypeStruct((NP, SCOLS), jnp.float32),
      jax.ShapeDtypeStruct((NP, SCOLS), jnp.float32),
  ]

  scratch = [
      pltpu.VMEM((16,), jnp.float32),
      pltpu.VMEM((CH,), jnp.int32),
      pltpu.VMEM((CH,), jnp.int32),
      pltpu.VMEM((CH, 16), jnp.float32),
      pltpu.VMEM((CH, 16), jnp.float32),
      pltpu.VMEM((CH, k_cols), jnp.float32),
      pltpu.VMEM((CH, SCOLS), jnp.float32),
      pltpu.VMEM((ZROWS, SCOLS), jnp.float32),
      pltpu.VMEM_SHARED((NP, SCOLS), jnp.float32),
      pltpu.SemaphoreType.DMA,
      pltpu.SemaphoreType.DMA,
      pltpu.SemaphoreType.DMA,
  ]

  def body(src_hbm, dst_hbm, ptab, xwa, xwb, cpad_hbm, acca_out, accb_out,
           cv_ref, sidx, didx, psrc, pdst, xwv, msg, zbuf, acc_sh,
           sem1, sem2, sem3):
    cid = lax.axis_index("c")
    sid = lax.axis_index("s")

    pltpu.sync_copy(cpad_hbm, cv_ref)
    cv = cv_ref[...]

    def zb_body(r, _):
      for j in range(SCOLS // 16):
        zbuf[r, j * 16:(j + 1) * 16] = jnp.zeros((16,), jnp.float32)
      return 0

    lax.fori_loop(0, ZROWS, zb_body, 0)

    row0 = sid * ROWS_PER_TILE

    def zacc_body(t, _):
      pltpu.sync_copy(zbuf, acc_sh.at[pl.ds(row0 + t * ZROWS, ZROWS)])
      return 0

    lax.fori_loop(0, ROWS_PER_TILE // ZROWS, zacc_body, 0)

    plsc.subcore_barrier()

    lanes = lax.broadcasted_iota(jnp.int32, (16,), 0)
    onehot0 = jnp.where(lanes == 0, 1.0, 0.0).astype(jnp.float32)
    perms = [lanes ^ st for st in (4, 2, 1)]
    ebase = sid * EDGES_PER_TILE

    def chunk_body(t, _):
      base = ebase + t * CH
      pltpu.sync_copy(src_hbm.at[pl.ds(base, CH)], sidx)
      pltpu.sync_copy(dst_hbm.at[pl.ds(base, CH)], didx)
      d1 = pltpu.async_copy(ptab.at[sidx], psrc, sem1)
      d2 = pltpu.async_copy(ptab.at[didx], pdst, sem2)

      @pl.when(cid == 0)
      def _ga():
        pltpu.async_copy(xwa.at[sidx], xwv, sem3)

      @pl.when(cid == 1)
      def _gb():
        pltpu.async_copy(xwb.at[sidx], xwv, sem3)

      d1.wait()
      d2.wait()
      pltpu.make_async_copy(xwa.at[sidx], xwv, sem3).wait()

      def edge_body(e, _):
        lg = psrc[e, :] - pdst[e, :] + cv
        m = lg
        for p_ in perms:
          m = jnp.maximum(m, m.at[p_].get(mode="promise_in_bounds"))
        ex = jnp.exp(lg - m)
        s2 = ex
        for p_ in perms:
          s2 = s2 + s2.at[p_].get(mode="promise_in_bounds")
        q = ex / s2
        for j in range(nv):
          mv = jnp.zeros((16,), jnp.float32)
          for h in range(HEADS):
            mv = mv + q[h] * xwv[e, pl.ds(h * chalf + j * 16, 16)]
          msg[e, pl.ds(j * 16, 16)] = mv
        if with_deg:
          msg[e, pl.ds(16, 16)] = onehot0
        return 0

      lax.fori_loop(0, CH, edge_body, 0)
      pltpu.sync_copy(msg, acc_sh.at[didx], add=True)
      return 0

    lax.fori_loop(0, nchunk, chunk_body, 0)

    plsc.subcore_barrier()

    rows = pl.ds(row0, ROWS_PER_TILE)

    @pl.when(cid == 0)
    def _wa():
      pltpu.sync_copy(acc_sh.at[rows], acca_out.at[rows])

    @pl.when(cid == 1)
    def _wb():
      pltpu.sync_copy(acc_sh.at[rows], accb_out.at[rows])

  return pl.kernel(
      body,
      out_type=out_type,
      mesh=mesh,
      scratch_types=scratch,
      compiler_params=pltpu.CompilerParams(use_tc_tiling_on_sc=False),
  )


def kernel(x, edge_index, batch, fc0_w, fc0_b, conv1_weight, conv1_u,
           conv1_c, conv1_bias, conv2_weight, conv2_u, conv2_c, conv2_bias,
           fc1_w, fc1_b):
  f32 = jnp.float32
  src = edge_index[0]
  dst = edge_index[1]

  w1r = conv1_weight.reshape(16, HEADS, 32)
  w1a = w1r[:, :, :16].reshape(16, 128)
  w1b = w1r[:, :, 16:].reshape(16, 128)
  u1p = jnp.concatenate([conv1_u, jnp.zeros((16, 8), f32)], axis=1)
  cpad1 = jnp.concatenate([conv1_c, jnp.zeros((8,), f32)])
  qc1 = jax.nn.softmax(conv1_c)
  qc1row = jnp.repeat(qc1, 16)[None, :]
  b1row = conv1_bias[None, :]

  w2r = conv2_weight.reshape(32, HEADS, 64)
  w2a = w2r[:, :, :32].reshape(32, 256)
  w2b = w2r[:, :, 32:].reshape(32, 256)
  u2p = jnp.concatenate([conv2_u, jnp.zeros((32, 8), f32)], axis=1)
  cpad2 = jnp.concatenate([conv2_c, jnp.zeros((8,), f32)])
  qc2 = jax.nn.softmax(conv2_c)
  qc2row = jnp.repeat(qc2, 32)[None, :]
  b2row = conv2_bias[None, :]

  fc0b = fc0_b[None, :]
  fc1wp = jnp.concatenate([fc1_w, jnp.zeros((64, 6), f32)], axis=1)
  fc1bp = jnp.concatenate([fc1_b, jnp.zeros((6,), f32)])[None, :]
  xp = jnp.pad(x, ((0, NP - N), (0, 0)))
  batch2d = jnp.pad(batch, (0, NP - N), constant_values=NUM_GRAPHS)[:, None]

  xw1a, xw1b, pt1 = _stage1(xp, fc0_w, fc0b, w1a, w1b, u1p)

  ek1 = _make_edge_kernel(16, 128, True)
  r1a, r1b = ek1(src, dst, pt1, xw1a, xw1b, cpad1)
  acc1a = r1a[:, :16]
  acc1b = r1b[:, :16]
  deg2d = r1a[:, 16:17]

  xw2a, xw2b, pt2 = _stage2(acc1a, acc1b, xw1a, xw1b, deg2d, qc1row, b1row,
                            w2a, w2b, u2p)

  ek2 = _make_edge_kernel(32, 256, False)
  acc2a, acc2b = ek2(src, dst, pt2, xw2a, xw2b, cpad2)

  outp = _stage3(acc2a, acc2b, xw2a, xw2b, deg2d, qc2row, b2row, batch2d,
                 fc1wp, fc1bp)
  return outp[:, :10]


# speedup vs baseline: 2.5513x; 1.2490x over previous
import functools

import jax
import jax.numpy as jnp
from jax import lax
from jax.experimental import pallas as pl
from jax.experimental.pallas import tpu as pltpu
from jax.experimental.pallas import tpu_sc as plsc

N = 50000
NP = 50176
E = 800000
HEADS = 8
NUM_GRAPHS = 16

ROW_BLK = 1024
NEG = -1e30

NUM_CORES = 2
NUM_SUBCORES = 16
ROWS_PER_TILE = NP // NUM_SUBCORES
ZROWS = 56


def _tc_grid_spec(n_in, n_out):
  pass


def _stage1_body(x_ref, w0_ref, b0_ref, wa_ref, wb_ref, up_ref,
                 xwa_ref, xwb_ref, pt_ref):
  h = jnp.maximum(
      jnp.dot(x_ref[...], w0_ref[...], preferred_element_type=jnp.float32)
      + b0_ref[...], 0.0)
  xwa_ref[...] = jnp.dot(h, wa_ref[...], preferred_element_type=jnp.float32)
  xwb_ref[...] = jnp.dot(h, wb_ref[...], preferred_element_type=jnp.float32)
  pt_ref[...] = jnp.dot(h, up_ref[...], preferred_element_type=jnp.float32)


def _stage1(x, w0, b0, wa, wb, up):
  nblk = NP // ROW_BLK
  rb = lambda i: (i, 0)
  whole = lambda i: (0, 0)
  return pl.pallas_call(
      _stage1_body,
      grid=(nblk,),
      in_specs=[
          pl.BlockSpec((ROW_BLK, 128), rb),
          pl.BlockSpec((128, 16), whole),
          pl.BlockSpec((1, 16), whole),
          pl.BlockSpec((16, 128), whole),
          pl.BlockSpec((16, 128), whole),
          pl.BlockSpec((16, 16), whole),
      ],
      out_specs=[
          pl.BlockSpec((ROW_BLK, 128), rb),
          pl.BlockSpec((ROW_BLK, 128), rb),
          pl.BlockSpec((ROW_BLK, 16), rb),
      ],
      out_shape=[
          jax.ShapeDtypeStruct((NP, 128), jnp.float32),
          jax.ShapeDtypeStruct((NP, 128), jnp.float32),
          jax.ShapeDtypeStruct((NP, 16), jnp.float32),
      ],
  )(x, w0, b0, wa, wb, up)


def _stage2_body(acca_ref, accb_ref, xwa_ref, xwb_ref, deg_ref, qc_ref,
                 b_ref, wa_ref, wb_ref, up_ref, xwa2_ref, xwb2_ref, pt2_ref,
                 *, chalf, heads):
  degf = deg_ref[...] + 1.0
  inv = 1.0 / degf
  selfa = (xwa_ref[...] * qc_ref[...]).reshape(-1, heads, chalf).sum(axis=1)
  selfb = (xwb_ref[...] * qc_ref[...]).reshape(-1, heads, chalf).sum(axis=1)
  ha = jnp.maximum((acca_ref[...] + selfa) * inv + b_ref[:, :chalf], 0.0)
  hb = jnp.maximum((accb_ref[...] + selfb) * inv + b_ref[:, chalf:], 0.0)
  h = jnp.concatenate([ha, hb], axis=1)
  xwa2_ref[...] = jnp.dot(h, wa_ref[...], preferred_element_type=jnp.float32)
  xwb2_ref[...] = jnp.dot(h, wb_ref[...], preferred_element_type=jnp.float32)
  pt2_ref[...] = jnp.dot(h, up_ref[...], preferred_element_type=jnp.float32)


def _stage2(acca, accb, xwa, xwb, deg2d, qcrow, brow, w2a, w2b, u2p):
  nblk = NP // ROW_BLK
  rb = lambda i: (i, 0)
  whole = lambda i: (0, 0)
  return pl.pallas_call(
      functools.partial(_stage2_body, chalf=16, heads=HEADS),
      grid=(nblk,),
      in_specs=[
          pl.BlockSpec((ROW_BLK, 16), rb),
          pl.BlockSpec((ROW_BLK, 16), rb),
          pl.BlockSpec((ROW_BLK, 128), rb),
          pl.BlockSpec((ROW_BLK, 128), rb),
          pl.BlockSpec((ROW_BLK, 1), rb),
          pl.BlockSpec((1, 128), whole),
          pl.BlockSpec((1, 32), whole),
          pl.BlockSpec((32, 256), whole),
          pl.BlockSpec((32, 256), whole),
          pl.BlockSpec((32, 16), whole),
      ],
      out_specs=[
          pl.BlockSpec((ROW_BLK, 256), rb),
          pl.BlockSpec((ROW_BLK, 256), rb),
          pl.BlockSpec((ROW_BLK, 16), rb),
      ],
      out_shape=[
          jax.ShapeDtypeStruct((NP, 256), jnp.float32),
          jax.ShapeDtypeStruct((NP, 256), jnp.float32),
          jax.ShapeDtypeStruct((NP, 16), jnp.float32),
      ],
  )(acca, accb, xwa, xwb, deg2d, qcrow, brow, w2a, w2b, u2p)


def _stage3_body(acca_ref, accb_ref, xwa_ref, xwb_ref, deg_ref, qc_ref,
                 b_ref, batch_ref, w1_ref, b1_ref, out_ref, pool_ref, cnt_ref,
                 *, chalf, heads, nblk):
  i = pl.program_id(0)

  @pl.when(i == 0)
  def _init():
    pool_ref[...] = jnp.zeros_like(pool_ref)
    cnt_ref[...] = jnp.zeros_like(cnt_ref)

  degf = deg_ref[...] + 1.0
  inv = 1.0 / degf
  selfa = (xwa_ref[...] * qc_ref[...]).reshape(-1, heads, chalf).sum(axis=1)
  selfb = (xwb_ref[...] * qc_ref[...]).reshape(-1, heads, chalf).sum(axis=1)
  ha = jnp.maximum((acca_ref[...] + selfa) * inv + b_ref[:, :chalf], 0.0)
  hb = jnp.maximum((accb_ref[...] + selfb) * inv + b_ref[:, chalf:], 0.0)
  h = jnp.concatenate([ha, hb], axis=1)
  bb = batch_ref[...]
  for g in range(NUM_GRAPHS):
    m = (bb == g).astype(jnp.float32)
    pool_ref[g:g + 1, :] += jnp.sum(h * m, axis=0, keepdims=True)
    cnt_ref[g:g + 1, :] += jnp.sum(m)

  @pl.when(i == nblk - 1)
  def _final():
    cnt = jnp.maximum(cnt_ref[:, :1], 1.0)
    pooled = pool_ref[...] / cnt
    out_ref[...] = (
        jnp.dot(pooled, w1_ref[...], preferred_element_type=jnp.float32)
        + b1_ref[...])


def _stage3(acca, accb, xwa, xwb, deg2d, qcrow, brow, batch2d, fc1_w, fc1b):
  nblk = NP // ROW_BLK
  rb = lambda i: (i, 0)
  whole = lambda i: (0, 0)
  return pl.pallas_call(
      functools.partial(_stage3_body, chalf=32, heads=HEADS, nblk=nblk),
      grid=(nblk,),
      in_specs=[
          pl.BlockSpec((ROW_BLK, 32), rb),
          pl.BlockSpec((ROW_BLK, 32), rb),
          pl.BlockSpec((ROW_BLK, 256), rb),
          pl.BlockSpec((ROW_BLK, 256), rb),
          pl.BlockSpec((ROW_BLK, 1), rb),
          pl.BlockSpec((1, 256), whole),
          pl.BlockSpec((1, 64), whole),
          pl.BlockSpec((ROW_BLK, 1), rb),
          pl.BlockSpec((64, 16), whole),
          pl.BlockSpec((1, 16), whole),
      ],
      out_specs=pl.BlockSpec((NUM_GRAPHS, 16), whole),
      out_shape=jax.ShapeDtypeStruct((NUM_GRAPHS, 16), jnp.float32),
      scratch_shapes=[
          pltpu.VMEM((NUM_GRAPHS, 64), jnp.float32),
          pltpu.VMEM((NUM_GRAPHS, 128), jnp.float32),
      ],
  )(acca, accb, xwa, xwb, deg2d, qcrow, brow, batch2d, fc1_w, fc1b)


SCOLS = 32


def _make_edge_kernel(chalf, k_cols, with_deg, ch, nchunk):
  nv = chalf // 16

  mesh = plsc.VectorSubcoreMesh(core_axis_name="c", subcore_axis_name="s")

  out_type = [
      jax.ShapeDtypeStruct((NP, SCOLS), jnp.float32),
      jax.ShapeDtypeStruct((NP, SCOLS), jnp.float32),
  ]

  scratch = [
      pltpu.VMEM((16,), jnp.float32),
      [pltpu.VMEM((2, ch), jnp.int32)] * 2,
      [pltpu.VMEM((ch, 16), jnp.float32)] * 2,
      [pltpu.VMEM((ch, 16), jnp.float32)] * 2,
      [pltpu.VMEM((ch, k_cols), jnp.float32)] * 2,
      [pltpu.VMEM((ch, SCOLS), jnp.float32)] * 2,
      pltpu.VMEM((ZROWS, SCOLS), jnp.float32),
      pltpu.VMEM_SHARED((NP, SCOLS), jnp.float32),
      [pltpu.SemaphoreType.DMA] * 2,
      [pltpu.SemaphoreType.DMA] * 2,
      [pltpu.SemaphoreType.DMA] * 2,
      [pltpu.SemaphoreType.DMA] * 2,
  ]

  def body(pk_hbm, ptab, xwa, xwb, cpad_hbm, acca_out, accb_out,
           cv_ref, ibuf, psrc, pdst, xwv, msg, zbuf, acc_sh,
           psem, dsem, xsem, ssem):
    cid = lax.axis_index("c")
    sid = lax.axis_index("s")

    pltpu.sync_copy(cpad_hbm, cv_ref)
    cv = cv_ref[...]

    def zb_body(r, _):
      for j in range(SCOLS // 16):
        zbuf[r, j * 16:(j + 1) * 16] = jnp.zeros((16,), jnp.float32)
      return 0

    lax.fori_loop(0, ZROWS, zb_body, 0)

    row0 = sid * ROWS_PER_TILE

    def zacc_body(t, _):
      pltpu.sync_copy(zbuf, acc_sh.at[pl.ds(row0 + t * ZROWS, ZROWS)])
      return 0

    lax.fori_loop(0, ROWS_PER_TILE // ZROWS, zacc_body, 0)

    plsc.subcore_barrier()

    lanes = lax.broadcasted_iota(jnp.int32, (16,), 0)
    onehot0 = jnp.where(lanes == 0, 1.0, 0.0).astype(jnp.float32)
    perms = [lanes ^ st for st in (4, 2, 1)]
    rbase = sid * nchunk

    def fire(b, c):
      pltpu.sync_copy(pk_hbm.at[rbase + c], ibuf[b])
      pltpu.async_copy(ptab.at[ibuf[b].at[0]], psrc[b], psem[b])
      pltpu.async_copy(ptab.at[ibuf[b].at[1]], pdst[b], dsem[b])

      @pl.when(cid == 0)
      def _ga():
        pltpu.async_copy(xwa.at[ibuf[b].at[0]], xwv[b], xsem[b])

      @pl.when(cid == 1)
      def _gb():
        pltpu.async_copy(xwb.at[ibuf[b].at[0]], xwv[b], xsem[b])

    def drain_gathers(b):
      pltpu.make_async_copy(ptab.at[ibuf[b].at[0]], psrc[b], psem[b]).wait()
      pltpu.make_async_copy(ptab.at[ibuf[b].at[1]], pdst[b], dsem[b]).wait()
      pltpu.make_async_copy(xwa.at[ibuf[b].at[0]], xwv[b], xsem[b]).wait()

    for b in (0, 1):
      fire(b, b)

    def super_body(t, _):
      c0 = t * 2
      for b in (0, 1):
        c = c0 + b

        @pl.when(c >= 2)
        def _ws():
          pltpu.make_async_copy(
              msg[b], acc_sh.at[ibuf[b].at[1]], ssem[b]).wait()

        drain_gathers(b)

        def edge_body(e, _):
          lg = psrc[b][e, :] - pdst[b][e, :] + cv
          m = lg
          for p_ in perms:
            m = jnp.maximum(m, m.at[p_].get(mode="promise_in_bounds"))
          ex = jnp.exp(lg - m)
          s2 = ex
          for p_ in perms:
            s2 = s2 + s2.at[p_].get(mode="promise_in_bounds")
          q = ex / s2
          for j in range(nv):
            mv = jnp.zeros((16,), jnp.float32)
            for h in range(HEADS):
              mv = mv + q[h] * xwv[b][e, pl.ds(h * chalf + j * 16, 16)]
            msg[b][e, pl.ds(j * 16, 16)] = mv
          if with_deg:
            msg[b][e, pl.ds(16, 16)] = onehot0
          return 0

        lax.fori_loop(0, ch, edge_body, 0)
        pltpu.async_copy(msg[b], acc_sh.at[ibuf[b].at[1]], ssem[b], add=True)
        fire(b, c + 2)
      return 0

    lax.fori_loop(0, nchunk // 2, super_body, 0)

    for b in (0, 1):
      pltpu.make_async_copy(msg[b], acc_sh.at[ibuf[b].at[1]], ssem[b]).wait()
      drain_gathers(b)

    plsc.subcore_barrier()

    rows = pl.ds(row0, ROWS_PER_TILE)

    @pl.when(cid == 0)
    def _wa():
      pltpu.sync_copy(acc_sh.at[rows], acca_out.at[rows])

    @pl.when(cid == 1)
    def _wb():
      pltpu.sync_copy(acc_sh.at[rows], accb_out.at[rows])

  return pl.kernel(
      body,
      out_type=out_type,
      mesh=mesh,
      scratch_types=scratch,
      compiler_params=pltpu.CompilerParams(use_tc_tiling_on_sc=False),
  )


def kernel(x, edge_index, batch, fc0_w, fc0_b, conv1_weight, conv1_u,
           conv1_c, conv1_bias, conv2_weight, conv2_u, conv2_c, conv2_bias,
           fc1_w, fc1_b):
  f32 = jnp.float32
  src = edge_index[0]
  dst = edge_index[1]

  w1r = conv1_weight.reshape(16, HEADS, 32)
  w1a = w1r[:, :, :16].reshape(16, 128)
  w1b = w1r[:, :, 16:].reshape(16, 128)
  u1p = jnp.concatenate([conv1_u, jnp.zeros((16, 8), f32)], axis=1)
  cpad1 = jnp.concatenate([conv1_c, jnp.zeros((8,), f32)])
  qc1 = jax.nn.softmax(conv1_c)
  qc1row = jnp.repeat(qc1, 16)[None, :]
  b1row = conv1_bias[None, :]

  w2r = conv2_weight.reshape(32, HEADS, 64)
  w2a = w2r[:, :, :32].reshape(32, 256)
  w2b = w2r[:, :, 32:].reshape(32, 256)
  u2p = jnp.concatenate([conv2_u, jnp.zeros((32, 8), f32)], axis=1)
  cpad2 = jnp.concatenate([conv2_c, jnp.zeros((8,), f32)])
  qc2 = jax.nn.softmax(conv2_c)
  qc2row = jnp.repeat(qc2, 32)[None, :]
  b2row = conv2_bias[None, :]

  fc0b = fc0_b[None, :]
  fc1wp = jnp.concatenate([fc1_w, jnp.zeros((64, 6), f32)], axis=1)
  fc1bp = jnp.concatenate([fc1_b, jnp.zeros((6,), f32)])[None, :]
  xp = jnp.pad(x, ((0, NP - N), (0, 0)))
  batch2d = jnp.pad(batch, (0, NP - N), constant_values=NUM_GRAPHS)[:, None]

  xw1a, xw1b, pt1 = _stage1(xp, fc0_w, fc0b, w1a, w1b, u1p)

  def _pack(ch, nchunk):
    per_tile = ch * nchunk
    s3 = jnp.pad(src.reshape(16, 50000), ((0, 0), (0, per_tile - 50000)),
                 constant_values=N).reshape(16 * nchunk, ch)
    d3 = jnp.pad(dst.reshape(16, 50000), ((0, 0), (0, per_tile - 50000)),
                 constant_values=N).reshape(16 * nchunk, ch)
    return jnp.pad(jnp.stack([s3, d3], axis=1), ((0, 8), (0, 0), (0, 0)),
                   constant_values=N)

  pk1 = _pack(64, 782)
  pk2 = _pack(40, 1250)

  ek1 = _make_edge_kernel(16, 128, True, 64, 782)
  r1a, r1b = ek1(pk1, pt1, xw1a, xw1b, cpad1)
  acc1a = r1a[:, :16]
  acc1b = r1b[:, :16]
  deg2d = r1a[:, 16:17]

  xw2a, xw2b, pt2 = _stage2(acc1a, acc1b, xw1a, xw1b, deg2d, qc1row, b1row,
                            w2a, w2b, u2p)

  ek2 = _make_edge_kernel(32, 256, False, 40, 1250)
  acc2a, acc2b = ek2(pk2, pt2, xw2a, xw2b, cpad2)

  outp = _stage3(acc2a, acc2b, xw2a, xw2b, deg2d, qc2row, b2row, batch2d,
                 fc1wp, fc1bp)
  return outp[:, :10]


# speedup vs baseline: 5.4331x; 2.1296x over previous
import functools

import jax
import jax.numpy as jnp
from jax import lax
from jax.experimental import pallas as pl
from jax.experimental.pallas import tpu as pltpu
from jax.experimental.pallas import tpu_sc as plsc

N = 50000
NP = 50176
E = 800000
HEADS = 8
NUM_GRAPHS = 16

ROW_BLK = 1024
NEG = -1e30

NUM_CORES = 2
NUM_SUBCORES = 16
ROWS_PER_TILE = NP // NUM_SUBCORES
ZROWS = 56


def _tc_grid_spec(n_in, n_out):
  pass


def _stage1_body(x_ref, w0_ref, b0_ref, wa_ref, wb_ref, up_ref,
                 xwa_ref, xwb_ref, pt_ref):
  h = jnp.maximum(
      jnp.dot(x_ref[...], w0_ref[...], preferred_element_type=jnp.float32)
      + b0_ref[...], 0.0)
  xwa_ref[...] = jnp.dot(h, wa_ref[...], preferred_element_type=jnp.float32)
  xwb_ref[...] = jnp.dot(h, wb_ref[...], preferred_element_type=jnp.float32)
  pt_ref[...] = jnp.dot(h, up_ref[...], preferred_element_type=jnp.float32)


def _stage1(x, w0, b0, wa, wb, up):
  nblk = NP // ROW_BLK
  rb = lambda i: (i, 0)
  whole = lambda i: (0, 0)
  return pl.pallas_call(
      _stage1_body,
      grid=(nblk,),
      in_specs=[
          pl.BlockSpec((ROW_BLK, 128), rb),
          pl.BlockSpec((128, 16), whole),
          pl.BlockSpec((1, 16), whole),
          pl.BlockSpec((16, 128), whole),
          pl.BlockSpec((16, 128), whole),
          pl.BlockSpec((16, 16), whole),
      ],
      out_specs=[
          pl.BlockSpec((ROW_BLK, 128), rb),
          pl.BlockSpec((ROW_BLK, 128), rb),
          pl.BlockSpec((ROW_BLK, 16), rb),
      ],
      out_shape=[
          jax.ShapeDtypeStruct((NP, 128), jnp.float32),
          jax.ShapeDtypeStruct((NP, 128), jnp.float32),
          jax.ShapeDtypeStruct((NP, 16), jnp.float32),
      ],
  )(x, w0, b0, wa, wb, up)


def _stage2_body(acca_ref, accb_ref, xwa_ref, xwb_ref, deg_ref, qc_ref,
                 b_ref, wa_ref, wb_ref, up_ref, xwa2_ref, xwb2_ref, pt2_ref,
                 *, chalf, heads):
  degf = deg_ref[...] + 1.0
  inv = 1.0 / degf
  selfa = (xwa_ref[...] * qc_ref[...]).reshape(-1, heads, chalf).sum(axis=1)
  selfb = (xwb_ref[...] * qc_ref[...]).reshape(-1, heads, chalf).sum(axis=1)
  ha = jnp.maximum((acca_ref[...] + selfa) * inv + b_ref[:, :chalf], 0.0)
  hb = jnp.maximum((accb_ref[...] + selfb) * inv + b_ref[:, chalf:], 0.0)
  h = jnp.concatenate([ha, hb], axis=1)
  xwa2_ref[...] = jnp.dot(h, wa_ref[...], preferred_element_type=jnp.float32)
  xwb2_ref[...] = jnp.dot(h, wb_ref[...], preferred_element_type=jnp.float32)
  pt2_ref[...] = jnp.dot(h, up_ref[...], preferred_element_type=jnp.float32)


def _stage2(acca, accb, xwa, xwb, deg2d, qcrow, brow, w2a, w2b, u2p):
  nblk = NP // ROW_BLK
  rb = lambda i: (i, 0)
  whole = lambda i: (0, 0)
  return pl.pallas_call(
      functools.partial(_stage2_body, chalf=16, heads=HEADS),
      grid=(nblk,),
      in_specs=[
          pl.BlockSpec((ROW_BLK, 16), rb),
          pl.BlockSpec((ROW_BLK, 16), rb),
          pl.BlockSpec((ROW_BLK, 128), rb),
          pl.BlockSpec((ROW_BLK, 128), rb),
          pl.BlockSpec((ROW_BLK, 1), rb),
          pl.BlockSpec((1, 128), whole),
          pl.BlockSpec((1, 32), whole),
          pl.BlockSpec((32, 256), whole),
          pl.BlockSpec((32, 256), whole),
          pl.BlockSpec((32, 16), whole),
      ],
      out_specs=[
          pl.BlockSpec((ROW_BLK, 256), rb),
          pl.BlockSpec((ROW_BLK, 256), rb),
          pl.BlockSpec((ROW_BLK, 16), rb),
      ],
      out_shape=[
          jax.ShapeDtypeStruct((NP, 256), jnp.float32),
          jax.ShapeDtypeStruct((NP, 256), jnp.float32),
          jax.ShapeDtypeStruct((NP, 16), jnp.float32),
      ],
  )(acca, accb, xwa, xwb, deg2d, qcrow, brow, w2a, w2b, u2p)


def _stage3_body(acca_ref, accb_ref, xwa_ref, xwb_ref, deg_ref, qc_ref,
                 b_ref, batch_ref, w1_ref, b1_ref, out_ref, pool_ref, cnt_ref,
                 *, chalf, heads, nblk):
  i = pl.program_id(0)

  @pl.when(i == 0)
  def _init():
    pool_ref[...] = jnp.zeros_like(pool_ref)
    cnt_ref[...] = jnp.zeros_like(cnt_ref)

  degf = deg_ref[...] + 1.0
  inv = 1.0 / degf
  selfa = (xwa_ref[...] * qc_ref[...]).reshape(-1, heads, chalf).sum(axis=1)
  selfb = (xwb_ref[...] * qc_ref[...]).reshape(-1, heads, chalf).sum(axis=1)
  ha = jnp.maximum((acca_ref[...] + selfa) * inv + b_ref[:, :chalf], 0.0)
  hb = jnp.maximum((accb_ref[...] + selfb) * inv + b_ref[:, chalf:], 0.0)
  h = jnp.concatenate([ha, hb], axis=1)
  bb = batch_ref[...]
  for g in range(NUM_GRAPHS):
    m = (bb == g).astype(jnp.float32)
    pool_ref[g:g + 1, :] += jnp.sum(h * m, axis=0, keepdims=True)
    cnt_ref[g:g + 1, :] += jnp.sum(m)

  @pl.when(i == nblk - 1)
  def _final():
    cnt = jnp.maximum(cnt_ref[:, :1], 1.0)
    pooled = pool_ref[...] / cnt
    out_ref[...] = (
        jnp.dot(pooled, w1_ref[...], preferred_element_type=jnp.float32)
        + b1_ref[...])


def _stage3(acca, accb, xwa, xwb, deg2d, qcrow, brow, batch2d, fc1_w, fc1b):
  nblk = NP // ROW_BLK
  rb = lambda i: (i, 0)
  whole = lambda i: (0, 0)
  return pl.pallas_call(
      functools.partial(_stage3_body, chalf=32, heads=HEADS, nblk=nblk),
      grid=(nblk,),
      in_specs=[
          pl.BlockSpec((ROW_BLK, 32), rb),
          pl.BlockSpec((ROW_BLK, 32), rb),
          pl.BlockSpec((ROW_BLK, 256), rb),
          pl.BlockSpec((ROW_BLK, 256), rb),
          pl.BlockSpec((ROW_BLK, 1), rb),
          pl.BlockSpec((1, 256), whole),
          pl.BlockSpec((1, 64), whole),
          pl.BlockSpec((ROW_BLK, 1), rb),
          pl.BlockSpec((64, 16), whole),
          pl.BlockSpec((1, 16), whole),
      ],
      out_specs=pl.BlockSpec((NUM_GRAPHS, 16), whole),
      out_shape=jax.ShapeDtypeStruct((NUM_GRAPHS, 16), jnp.float32),
      scratch_shapes=[
          pltpu.VMEM((NUM_GRAPHS, 64), jnp.float32),
          pltpu.VMEM((NUM_GRAPHS, 128), jnp.float32),
      ],
  )(acca, accb, xwa, xwb, deg2d, qcrow, brow, batch2d, fc1_w, fc1b)


SCOLS = 32


def _make_edge_kernel(chalf, k_cols, with_deg, ch, nchunk):
  nv = chalf // 16

  mesh = plsc.VectorSubcoreMesh(core_axis_name="c", subcore_axis_name="s")

  out_type = [
      jax.ShapeDtypeStruct((NP, SCOLS), jnp.float32),
      jax.ShapeDtypeStruct((NP, SCOLS), jnp.float32),
  ]

  scratch = [
      pltpu.VMEM((16,), jnp.float32),
      [pltpu.VMEM((2, ch), jnp.int32)] * 2,
      [pltpu.VMEM((ch, 16), jnp.float32)] * 2,
      [pltpu.VMEM((ch, 16), jnp.float32)] * 2,
      [pltpu.VMEM((ch, k_cols), jnp.float32)] * 2,
      [pltpu.VMEM((ch, SCOLS), jnp.float32)] * 2,
      pltpu.VMEM((ZROWS, SCOLS), jnp.float32),
      pltpu.VMEM_SHARED((NP, SCOLS), jnp.float32),
      [pltpu.SemaphoreType.DMA] * 2,
      [pltpu.SemaphoreType.DMA] * 2,
      [pltpu.SemaphoreType.DMA] * 2,
      [pltpu.SemaphoreType.DMA] * 2,
  ]

  def body(pk_hbm, ptab, xwa, xwb, cpad_hbm, acca_out, accb_out,
           cv_ref, ibuf, psrc, pdst, xwv, msg, zbuf, acc_sh,
           psem, dsem, xsem, ssem):
    cid = lax.axis_index("c")
    sid = lax.axis_index("s")

    pltpu.sync_copy(cpad_hbm, cv_ref)
    cv = cv_ref[...]

    def zb_body(r, _):
      for j in range(SCOLS // 16):
        zbuf[r, j * 16:(j + 1) * 16] = jnp.zeros((16,), jnp.float32)
      return 0

    lax.fori_loop(0, ZROWS, zb_body, 0)

    row0 = sid * ROWS_PER_TILE

    def zacc_body(t, _):
      pltpu.sync_copy(zbuf, acc_sh.at[pl.ds(row0 + t * ZROWS, ZROWS)])
      return 0

    lax.fori_loop(0, ROWS_PER_TILE // ZROWS, zacc_body, 0)

    plsc.subcore_barrier()

    lanes = lax.broadcasted_iota(jnp.int32, (16,), 0)
    onehot0 = jnp.where(lanes == 0, 1.0, 0.0).astype(jnp.float32)
    perms = [lanes ^ st for st in (4, 2, 1)]
    hbcast = [jnp.full((16,), h, jnp.int32) for h in range(HEADS)]
    rbase = sid * nchunk

    def fire(b, c):
      pltpu.sync_copy(pk_hbm.at[rbase + c], ibuf[b])
      pltpu.async_copy(ptab.at[ibuf[b].at[0]], psrc[b], psem[b])
      pltpu.async_copy(ptab.at[ibuf[b].at[1]], pdst[b], dsem[b])

      @pl.when(cid == 0)
      def _ga():
        pltpu.async_copy(xwa.at[ibuf[b].at[0]], xwv[b], xsem[b])

      @pl.when(cid == 1)
      def _gb():
        pltpu.async_copy(xwb.at[ibuf[b].at[0]], xwv[b], xsem[b])

    def drain_gathers(b):
      pltpu.make_async_copy(ptab.at[ibuf[b].at[0]], psrc[b], psem[b]).wait()
      pltpu.make_async_copy(ptab.at[ibuf[b].at[1]], pdst[b], dsem[b]).wait()
      pltpu.make_async_copy(xwa.at[ibuf[b].at[0]], xwv[b], xsem[b]).wait()

    for b in (0, 1):
      fire(b, b)

    def super_body(t, _):
      c0 = t * 2
      for b in (0, 1):
        c = c0 + b

        @pl.when(c >= 2)
        def _ws():
          pltpu.make_async_copy(
              msg[b], acc_sh.at[ibuf[b].at[1]], ssem[b]).wait()

        drain_gathers(b)

        @plsc.parallel_loop(0, ch, 1, unroll=4)
        def edge_body(e):
          lg = psrc[b][e, :] - pdst[b][e, :] + cv
          m = lg
          for p_ in perms:
            m = jnp.maximum(m, m.at[p_].get(mode="promise_in_bounds"))
          ex = jnp.exp(lg - m)
          s2 = ex
          for p_ in perms:
            s2 = s2 + s2.at[p_].get(mode="promise_in_bounds")
          q = ex / s2
          qb = [q.at[hb_].get(mode="promise_in_bounds") for hb_ in hbcast]
          for j in range(nv):
            pr = [qb[h] * xwv[b][e, pl.ds(h * chalf + j * 16, 16)]
                  for h in range(HEADS)]
            while len(pr) > 1:
              pr = [pr[2 * i_] + pr[2 * i_ + 1] for i_ in range(len(pr) // 2)]
            msg[b][e, pl.ds(j * 16, 16)] = pr[0]
          if with_deg:
            msg[b][e, pl.ds(16, 16)] = onehot0
        pltpu.async_copy(msg[b], acc_sh.at[ibuf[b].at[1]], ssem[b], add=True)
        fire(b, c + 2)
      return 0

    lax.fori_loop(0, nchunk // 2, super_body, 0)

    for b in (0, 1):
      pltpu.make_async_copy(msg[b], acc_sh.at[ibuf[b].at[1]], ssem[b]).wait()
      drain_gathers(b)

    plsc.subcore_barrier()

    rows = pl.ds(row0, ROWS_PER_TILE)

    @pl.when(cid == 0)
    def _wa():
      pltpu.sync_copy(acc_sh.at[rows], acca_out.at[rows])

    @pl.when(cid == 1)
    def _wb():
      pltpu.sync_copy(acc_sh.at[rows], accb_out.at[rows])

  return pl.kernel(
      body,
      out_type=out_type,
      mesh=mesh,
      scratch_types=scratch,
      compiler_params=pltpu.CompilerParams(use_tc_tiling_on_sc=False),
  )


def kernel(x, edge_index, batch, fc0_w, fc0_b, conv1_weight, conv1_u,
           conv1_c, conv1_bias, conv2_weight, conv2_u, conv2_c, conv2_bias,
           fc1_w, fc1_b):
  f32 = jnp.float32
  src = edge_index[0]
  dst = edge_index[1]

  w1r = conv1_weight.reshape(16, HEADS, 32)
  w1a = w1r[:, :, :16].reshape(16, 128)
  w1b = w1r[:, :, 16:].reshape(16, 128)
  u1p = jnp.concatenate([conv1_u, jnp.zeros((16, 8), f32)], axis=1)
  cpad1 = jnp.concatenate([conv1_c, jnp.zeros((8,), f32)])
  qc1 = jax.nn.softmax(conv1_c)
  qc1row = jnp.repeat(qc1, 16)[None, :]
  b1row = conv1_bias[None, :]

  w2r = conv2_weight.reshape(32, HEADS, 64)
  w2a = w2r[:, :, :32].reshape(32, 256)
  w2b = w2r[:, :, 32:].reshape(32, 256)
  u2p = jnp.concatenate([conv2_u, jnp.zeros((32, 8), f32)], axis=1)
  cpad2 = jnp.concatenate([conv2_c, jnp.zeros((8,), f32)])
  qc2 = jax.nn.softmax(conv2_c)
  qc2row = jnp.repeat(qc2, 32)[None, :]
  b2row = conv2_bias[None, :]

  fc0b = fc0_b[None, :]
  fc1wp = jnp.concatenate([fc1_w, jnp.zeros((64, 6), f32)], axis=1)
  fc1bp = jnp.concatenate([fc1_b, jnp.zeros((6,), f32)])[None, :]
  xp = jnp.pad(x, ((0, NP - N), (0, 0)))
  batch2d = jnp.pad(batch, (0, NP - N), constant_values=NUM_GRAPHS)[:, None]

  xw1a, xw1b, pt1 = _stage1(xp, fc0_w, fc0b, w1a, w1b, u1p)

  def _pack(ch, nchunk):
    per_tile = ch * nchunk
    s3 = jnp.pad(src.reshape(16, 50000), ((0, 0), (0, per_tile - 50000)),
                 constant_values=N).reshape(16 * nchunk, ch)
    d3 = jnp.pad(dst.reshape(16, 50000), ((0, 0), (0, per_tile - 50000)),
                 constant_values=N).reshape(16 * nchunk, ch)
    return jnp.pad(jnp.stack([s3, d3], axis=1), ((0, 8), (0, 0), (0, 0)),
                   constant_values=N)

  pk1 = _pack(64, 782)
  pk2 = _pack(40, 1250)

  ek1 = _make_edge_kernel(16, 128, True, 64, 782)
  r1a, r1b = ek1(pk1, pt1, xw1a, xw1b, cpad1)
  acc1a = r1a[:, :16]
  acc1b = r1b[:, :16]
  deg2d = r1a[:, 16:17]

  xw2a, xw2b, pt2 = _stage2(acc1a, acc1b, xw1a, xw1b, deg2d, qc1row, b1row,
                            w2a, w2b, u2p)

  ek2 = _make_edge_kernel(32, 256, False, 40, 1250)
  acc2a, acc2b = ek2(pk2, pt2, xw2a, xw2b, cpad2)

  outp = _stage3(acc2a, acc2b, xw2a, xw2b, deg2d, qc2row, b2row, batch2d,
                 fc1wp, fc1bp)
  return outp[:, :10]


# speedup vs baseline: 5.7953x; 1.0667x over previous
import functools

import jax
import jax.numpy as jnp
from jax import lax
from jax.experimental import pallas as pl
from jax.experimental.pallas import tpu as pltpu
from jax.experimental.pallas import tpu_sc as plsc

N = 50000
NP = 50176
E = 800000
HEADS = 8
NUM_GRAPHS = 16

ROW_BLK = 1024
NEG = -1e30

NUM_CORES = 2
NUM_SUBCORES = 16
ROWS_PER_TILE = NP // NUM_SUBCORES
ZROWS = 56


def _tc_grid_spec(n_in, n_out):
  pass


def _stage1_body(x_ref, w0_ref, b0_ref, wa_ref, wb_ref, up_ref,
                 xwa_ref, xwb_ref, pt_ref):
  h = jnp.maximum(
      jnp.dot(x_ref[...], w0_ref[...], preferred_element_type=jnp.float32)
      + b0_ref[...], 0.0)
  xwa_ref[...] = jnp.dot(h, wa_ref[...], preferred_element_type=jnp.float32)
  xwb_ref[...] = jnp.dot(h, wb_ref[...], preferred_element_type=jnp.float32)
  pt_ref[...] = jnp.dot(h, up_ref[...], preferred_element_type=jnp.float32)


def _stage1(x, w0, b0, wa, wb, up):
  nblk = NP // ROW_BLK
  rb = lambda i: (i, 0)
  whole = lambda i: (0, 0)
  return pl.pallas_call(
      _stage1_body,
      grid=(nblk,),
      in_specs=[
          pl.BlockSpec((ROW_BLK, 128), rb),
          pl.BlockSpec((128, 16), whole),
          pl.BlockSpec((1, 16), whole),
          pl.BlockSpec((16, 128), whole),
          pl.BlockSpec((16, 128), whole),
          pl.BlockSpec((16, 16), whole),
      ],
      out_specs=[
          pl.BlockSpec((ROW_BLK, 128), rb),
          pl.BlockSpec((ROW_BLK, 128), rb),
          pl.BlockSpec((ROW_BLK, 16), rb),
      ],
      out_shape=[
          jax.ShapeDtypeStruct((NP, 128), jnp.float32),
          jax.ShapeDtypeStruct((NP, 128), jnp.float32),
          jax.ShapeDtypeStruct((NP, 16), jnp.float32),
      ],
  )(x, w0, b0, wa, wb, up)


def _stage2_body(acca_ref, accb_ref, xwa_ref, xwb_ref, deg_ref, qc_ref,
                 b_ref, wa_ref, wb_ref, up_ref, xwa2_ref, xwb2_ref, pt2_ref,
                 *, chalf, heads):
  degf = deg_ref[...] + 1.0
  inv = 1.0 / degf
  selfa = (xwa_ref[...] * qc_ref[...]).reshape(-1, heads, chalf).sum(axis=1)
  selfb = (xwb_ref[...] * qc_ref[...]).reshape(-1, heads, chalf).sum(axis=1)
  ha = jnp.maximum((acca_ref[...] + selfa) * inv + b_ref[:, :chalf], 0.0)
  hb = jnp.maximum((accb_ref[...] + selfb) * inv + b_ref[:, chalf:], 0.0)
  h = jnp.concatenate([ha, hb], axis=1)
  xwa2_ref[...] = jnp.dot(h, wa_ref[...], preferred_element_type=jnp.float32)
  xwb2_ref[...] = jnp.dot(h, wb_ref[...], preferred_element_type=jnp.float32)
  pt2_ref[...] = jnp.dot(h, up_ref[...], preferred_element_type=jnp.float32)


def _stage2(acca, accb, xwa, xwb, deg2d, qcrow, brow, w2a, w2b, u2p):
  nblk = NP // ROW_BLK
  rb = lambda i: (i, 0)
  whole = lambda i: (0, 0)
  return pl.pallas_call(
      functools.partial(_stage2_body, chalf=16, heads=HEADS),
      grid=(nblk,),
      in_specs=[
          pl.BlockSpec((ROW_BLK, 16), rb),
          pl.BlockSpec((ROW_BLK, 16), rb),
          pl.BlockSpec((ROW_BLK, 128), rb),
          pl.BlockSpec((ROW_BLK, 128), rb),
          pl.BlockSpec((ROW_BLK, 1), rb),
          pl.BlockSpec((1, 128), whole),
          pl.BlockSpec((1, 32), whole),
          pl.BlockSpec((32, 256), whole),
          pl.BlockSpec((32, 256), whole),
          pl.BlockSpec((32, 16), whole),
      ],
      out_specs=[
          pl.BlockSpec((ROW_BLK, 256), rb),
          pl.BlockSpec((ROW_BLK, 256), rb),
          pl.BlockSpec((ROW_BLK, 16), rb),
      ],
      out_shape=[
          jax.ShapeDtypeStruct((NP, 256), jnp.float32),
          jax.ShapeDtypeStruct((NP, 256), jnp.float32),
          jax.ShapeDtypeStruct((NP, 16), jnp.float32),
      ],
  )(acca, accb, xwa, xwb, deg2d, qcrow, brow, w2a, w2b, u2p)


def _stage3_body(acca_ref, accb_ref, xwa_ref, xwb_ref, deg_ref, qc_ref,
                 b_ref, batch_ref, w1_ref, b1_ref, out_ref, pool_ref, cnt_ref,
                 *, chalf, heads, nblk):
  i = pl.program_id(0)

  @pl.when(i == 0)
  def _init():
    pool_ref[...] = jnp.zeros_like(pool_ref)
    cnt_ref[...] = jnp.zeros_like(cnt_ref)

  degf = deg_ref[...] + 1.0
  inv = 1.0 / degf
  selfa = (xwa_ref[...] * qc_ref[...]).reshape(-1, heads, chalf).sum(axis=1)
  selfb = (xwb_ref[...] * qc_ref[...]).reshape(-1, heads, chalf).sum(axis=1)
  ha = jnp.maximum((acca_ref[...] + selfa) * inv + b_ref[:, :chalf], 0.0)
  hb = jnp.maximum((accb_ref[...] + selfb) * inv + b_ref[:, chalf:], 0.0)
  h = jnp.concatenate([ha, hb], axis=1)
  bb = batch_ref[...]
  for g in range(NUM_GRAPHS):
    m = (bb == g).astype(jnp.float32)
    pool_ref[g:g + 1, :] += jnp.sum(h * m, axis=0, keepdims=True)
    cnt_ref[g:g + 1, :] += jnp.sum(m)

  @pl.when(i == nblk - 1)
  def _final():
    cnt = jnp.maximum(cnt_ref[:, :1], 1.0)
    pooled = pool_ref[...] / cnt
    out_ref[...] = (
        jnp.dot(pooled, w1_ref[...], preferred_element_type=jnp.float32)
        + b1_ref[...])


def _stage3(acca, accb, xwa, xwb, deg2d, qcrow, brow, batch2d, fc1_w, fc1b):
  nblk = NP // ROW_BLK
  rb = lambda i: (i, 0)
  whole = lambda i: (0, 0)
  return pl.pallas_call(
      functools.partial(_stage3_body, chalf=32, heads=HEADS, nblk=nblk),
      grid=(nblk,),
      in_specs=[
          pl.BlockSpec((ROW_BLK, 32), rb),
          pl.BlockSpec((ROW_BLK, 32), rb),
          pl.BlockSpec((ROW_BLK, 256), rb),
          pl.BlockSpec((ROW_BLK, 256), rb),
          pl.BlockSpec((ROW_BLK, 1), rb),
          pl.BlockSpec((1, 256), whole),
          pl.BlockSpec((1, 64), whole),
          pl.BlockSpec((ROW_BLK, 1), rb),
          pl.BlockSpec((64, 16), whole),
          pl.BlockSpec((1, 16), whole),
      ],
      out_specs=pl.BlockSpec((NUM_GRAPHS, 16), whole),
      out_shape=jax.ShapeDtypeStruct((NUM_GRAPHS, 16), jnp.float32),
      scratch_shapes=[
          pltpu.VMEM((NUM_GRAPHS, 64), jnp.float32),
          pltpu.VMEM((NUM_GRAPHS, 128), jnp.float32),
      ],
  )(acca, accb, xwa, xwb, deg2d, qcrow, brow, batch2d, fc1_w, fc1b)


SCOLS = 32


def _make_edge_kernel(chalf, k_cols, with_deg, ch, nchunk):
  nv = chalf // 16

  mesh = plsc.VectorSubcoreMesh(core_axis_name="c", subcore_axis_name="s")

  out_type = [
      jax.ShapeDtypeStruct((NP, SCOLS), jnp.float32),
      jax.ShapeDtypeStruct((NP, SCOLS), jnp.float32),
  ]

  scratch = [
      pltpu.VMEM((16,), jnp.float32),
      [pltpu.VMEM((2, ch), jnp.int32)] * 4,
      [pltpu.VMEM((ch, 16), jnp.float32)] * 2,
      [pltpu.VMEM((ch, 16), jnp.float32)] * 2,
      [pltpu.VMEM((ch, k_cols), jnp.float32)] * 2,
      [pltpu.VMEM((ch, SCOLS), jnp.float32)] * 2,
      pltpu.VMEM((ZROWS, SCOLS), jnp.float32),
      pltpu.VMEM_SHARED((NP, SCOLS), jnp.float32),
      [pltpu.SemaphoreType.DMA] * 2,
      [pltpu.SemaphoreType.DMA] * 2,
      [pltpu.SemaphoreType.DMA] * 2,
      [pltpu.SemaphoreType.DMA] * 2,
      [pltpu.SemaphoreType.DMA] * 2,
  ]

  def body(pk_hbm, ptab, xwa, xwb, cpad_hbm, acca_out, accb_out,
           cv_ref, ibuf, psrc, pdst, xwv, msg, zbuf, acc_sh,
           psem, dsem, xsem, ssem, isem):
    cid = lax.axis_index("c")
    sid = lax.axis_index("s")

    pltpu.sync_copy(cpad_hbm, cv_ref)
    cv = cv_ref[...]

    def zb_body(r, _):
      for j in range(SCOLS // 16):
        zbuf[r, j * 16:(j + 1) * 16] = jnp.zeros((16,), jnp.float32)
      return 0

    lax.fori_loop(0, ZROWS, zb_body, 0)

    row0 = sid * ROWS_PER_TILE

    def zacc_body(t, _):
      pltpu.sync_copy(zbuf, acc_sh.at[pl.ds(row0 + t * ZROWS, ZROWS)])
      return 0

    lax.fori_loop(0, ROWS_PER_TILE // ZROWS, zacc_body, 0)

    plsc.subcore_barrier()

    lanes = lax.broadcasted_iota(jnp.int32, (16,), 0)
    onehot0 = jnp.where(lanes == 0, 1.0, 0.0).astype(jnp.float32)
    perms = [lanes ^ st for st in (4, 2, 1)]
    hbcast = [jnp.full((16,), h, jnp.int32) for h in range(HEADS)]
    rbase = sid * nchunk

    def fire_gathers(b, ii):
      pltpu.async_copy(ptab.at[ibuf[ii].at[0]], psrc[b], psem[b])
      pltpu.async_copy(ptab.at[ibuf[ii].at[1]], pdst[b], dsem[b])

      @pl.when(cid == 0)
      def _ga():
        pltpu.async_copy(xwa.at[ibuf[ii].at[0]], xwv[b], xsem[b])

      @pl.when(cid == 1)
      def _gb():
        pltpu.async_copy(xwb.at[ibuf[ii].at[0]], xwv[b], xsem[b])

    def drain_gathers(b, ii):
      pltpu.make_async_copy(ptab.at[ibuf[ii].at[0]], psrc[b], psem[b]).wait()
      pltpu.make_async_copy(ptab.at[ibuf[ii].at[1]], pdst[b], dsem[b]).wait()
      pltpu.make_async_copy(xwa.at[ibuf[ii].at[0]], xwv[b], xsem[b]).wait()

    for b in (0, 1):
      pltpu.sync_copy(pk_hbm.at[rbase + b], ibuf[b])
      fire_gathers(b, b)

    def super_body(t, _):
      c0 = t * 4
      for k in range(4):
        b = k % 2
        ii = k
        iin = (k + 2) % 4
        c = c0 + k

        @pl.when(c >= 2)
        def _ws():
          pltpu.make_async_copy(
              msg[b], acc_sh.at[ibuf[iin].at[1]], ssem[b]).wait()

        pltpu.async_copy(pk_hbm.at[rbase + c + 2], ibuf[iin], isem[b])

        drain_gathers(b, ii)

        @plsc.parallel_loop(0, ch, 1, unroll=4)
        def edge_body(e):
          lg = psrc[b][e, :] - pdst[b][e, :] + cv
          m = lg
          for p_ in perms:
            m = jnp.maximum(m, m.at[p_].get(mode="promise_in_bounds"))
          ex = jnp.exp(lg - m)
          s2 = ex
          for p_ in perms:
            s2 = s2 + s2.at[p_].get(mode="promise_in_bounds")
          q = ex / s2
          qb = [q.at[hb_].get(mode="promise_in_bounds") for hb_ in hbcast]
          for j in range(nv):
            pr = [qb[h] * xwv[b][e, pl.ds(h * chalf + j * 16, 16)]
                  for h in range(HEADS)]
            while len(pr) > 1:
              pr = [pr[2 * i_] + pr[2 * i_ + 1] for i_ in range(len(pr) // 2)]
            msg[b][e, pl.ds(j * 16, 16)] = pr[0]
          if with_deg:
            msg[b][e, pl.ds(16, 16)] = onehot0
        pltpu.async_copy(msg[b], acc_sh.at[ibuf[ii].at[1]], ssem[b], add=True)
        pltpu.make_async_copy(
            pk_hbm.at[rbase + c + 2], ibuf[iin], isem[b]).wait()
        fire_gathers(b, iin)
      return 0

    lax.fori_loop(0, nchunk // 4, super_body, 0)

    for b in (0, 1):
      pltpu.make_async_copy(
          msg[b], acc_sh.at[ibuf[2 + b].at[1]], ssem[b]).wait()
      drain_gathers(b, b)

    plsc.subcore_barrier()

    rows = pl.ds(row0, ROWS_PER_TILE)

    @pl.when(cid == 0)
    def _wa():
      pltpu.sync_copy(acc_sh.at[rows], acca_out.at[rows])

    @pl.when(cid == 1)
    def _wb():
      pltpu.sync_copy(acc_sh.at[rows], accb_out.at[rows])

  return pl.kernel(
      body,
      out_type=out_type,
      mesh=mesh,
      scratch_types=scratch,
      compiler_params=pltpu.CompilerParams(use_tc_tiling_on_sc=False),
  )


def kernel(x, edge_index, batch, fc0_w, fc0_b, conv1_weight, conv1_u,
           conv1_c, conv1_bias, conv2_weight, conv2_u, conv2_c, conv2_bias,
           fc1_w, fc1_b):
  f32 = jnp.float32
  src = edge_index[0]
  dst = edge_index[1]

  w1r = conv1_weight.reshape(16, HEADS, 32)
  w1a = w1r[:, :, :16].reshape(16, 128)
  w1b = w1r[:, :, 16:].reshape(16, 128)
  u1p = jnp.concatenate([conv1_u, jnp.zeros((16, 8), f32)], axis=1)
  cpad1 = jnp.concatenate([conv1_c, jnp.zeros((8,), f32)])
  qc1 = jax.nn.softmax(conv1_c)
  qc1row = jnp.repeat(qc1, 16)[None, :]
  b1row = conv1_bias[None, :]

  w2r = conv2_weight.reshape(32, HEADS, 64)
  w2a = w2r[:, :, :32].reshape(32, 256)
  w2b = w2r[:, :, 32:].reshape(32, 256)
  u2p = jnp.concatenate([conv2_u, jnp.zeros((32, 8), f32)], axis=1)
  cpad2 = jnp.concatenate([conv2_c, jnp.zeros((8,), f32)])
  qc2 = jax.nn.softmax(conv2_c)
  qc2row = jnp.repeat(qc2, 32)[None, :]
  b2row = conv2_bias[None, :]

  fc0b = fc0_b[None, :]
  fc1wp = jnp.concatenate([fc1_w, jnp.zeros((64, 6), f32)], axis=1)
  fc1bp = jnp.concatenate([fc1_b, jnp.zeros((6,), f32)])[None, :]
  xp = jnp.pad(x, ((0, NP - N), (0, 0)))
  batch2d = jnp.pad(batch, (0, NP - N), constant_values=NUM_GRAPHS)[:, None]

  xw1a, xw1b, pt1 = _stage1(xp, fc0_w, fc0b, w1a, w1b, u1p)

  def _pack(ch, nchunk):
    per_tile = ch * nchunk
    s3 = jnp.pad(src.reshape(16, 50000), ((0, 0), (0, per_tile - 50000)),
                 constant_values=N).reshape(16 * nchunk, ch)
    d3 = jnp.pad(dst.reshape(16, 50000), ((0, 0), (0, per_tile - 50000)),
                 constant_values=N).reshape(16 * nchunk, ch)
    return jnp.pad(jnp.stack([s3, d3], axis=1), ((0, 8), (0, 0), (0, 0)),
                   constant_values=N)

  pk1 = _pack(64, 784)
  pk2 = _pack(40, 1252)

  ek1 = _make_edge_kernel(16, 128, True, 64, 784)
  r1a, r1b = ek1(pk1, pt1, xw1a, xw1b, cpad1)
  acc1a = r1a[:, :16]
  acc1b = r1b[:, :16]
  deg2d = r1a[:, 16:17]

  xw2a, xw2b, pt2 = _stage2(acc1a, acc1b, xw1a, xw1b, deg2d, qc1row, b1row,
                            w2a, w2b, u2p)

  ek2 = _make_edge_kernel(32, 256, False, 40, 1252)
  acc2a, acc2b = ek2(pk2, pt2, xw2a, xw2b, cpad2)

  outp = _stage3(acc2a, acc2b, xw2a, xw2b, deg2d, qc2row, b2row, batch2d,
                 fc1wp, fc1bp)
  return outp[:, :10]
